# Initial kernel scaffold; baseline (speedup 1.0000x reference)
import jax
import jax.numpy as jnp
from jax.experimental import pallas as pl


def kernel(inputs, edge_index, layer_id, n_layers, W_self, W_neigh, b):
    raise NotImplementedError("write your pallas kernel here")


# speedup vs baseline: 6.3258x; 6.3258x over previous
import functools

import jax
import jax.numpy as jnp
from jax import lax
from jax.experimental import pallas as pl
from jax.experimental.pallas import tpu as pltpu
from jax.experimental.pallas import tpu_sc as plsc

NC = 2
NS = 16
NW = NC * NS
L = 16
CHUNK = 128


def _sc_aggregate(n_pad, d, n_chunks):
    rows_per_tile = n_pad // NS
    mesh = plsc.VectorSubcoreMesh(core_axis_name="c", subcore_axis_name="s",
                                  num_cores=NC, num_subcores=NS)

    @functools.partial(
        pl.kernel,
        out_type=(
            jax.ShapeDtypeStruct((NC, n_pad, d), jnp.float32),
            jax.ShapeDtypeStruct((NC, n_pad), jnp.float32),
        ),
        mesh=mesh,
        scratch_types=(
            pltpu.VMEM((n_chunks, CHUNK), jnp.int32),
            pltpu.VMEM((n_chunks, CHUNK), jnp.int32),
            pltpu.VMEM((CHUNK, d), jnp.float32),
            pltpu.VMEM((CHUNK,), jnp.float32),
            pltpu.VMEM((rows_per_tile,), jnp.float32),
            pltpu.VMEM_SHARED((n_pad, d), jnp.float32),
            pltpu.VMEM_SHARED((n_pad,), jnp.float32),
            pltpu.SemaphoreType.DMA,
        ),
    )
    def body(src_hbm, dst_hbm, x_hbm, agg_out, deg_out,
             src_v, dst_v, rows_v, ones_v, zdeg_v, agg_sh, deg_sh, sem):
        c = lax.axis_index("c")
        s = lax.axis_index("s")
        wid = c * NS + s
        r0 = s * rows_per_tile

        zeros16 = jnp.zeros((L,), jnp.float32)
        ones16 = jnp.ones((L,), jnp.float32)

        def zero_row(i, _):
            def zero_col(j, _):
                rows_v[i, pl.ds(j * L, L)] = zeros16
                return 0
            return lax.fori_loop(0, d // L, zero_col, 0)
        lax.fori_loop(0, CHUNK, zero_row, 0)

        for k in range(CHUNK // L):
            ones_v[pl.ds(k * L, L)] = ones16

        def zero_deg(i, _):
            zdeg_v[pl.ds(i * L, L)] = zeros16
            return 0
        lax.fori_loop(0, rows_per_tile // L, zero_deg, 0)

        for k in range(rows_per_tile // CHUNK):
            pltpu.sync_copy(rows_v, agg_sh.at[pl.ds(r0 + k * CHUNK, CHUNK)])
        pltpu.sync_copy(zdeg_v, deg_sh.at[pl.ds(r0, rows_per_tile)])

        pltpu.sync_copy(src_hbm.at[wid], src_v)
        pltpu.sync_copy(dst_hbm.at[wid], dst_v)

        plsc.subcore_barrier()

        def edge_chunk(j, _):
            pltpu.async_copy(x_hbm.at[src_v.at[j]], rows_v, sem).wait()
            pltpu.sync_copy(rows_v, agg_sh.at[dst_v.at[j]], add=True)
            pltpu.sync_copy(ones_v, deg_sh.at[dst_v.at[j]], add=True)
            return 0
        lax.fori_loop(0, n_chunks, edge_chunk, 0)

        plsc.subcore_barrier()

        pltpu.sync_copy(agg_sh.at[pl.ds(r0, rows_per_tile)],
                        agg_out.at[c, pl.ds(r0, rows_per_tile)])
        pltpu.sync_copy(deg_sh.at[pl.ds(r0, rows_per_tile)],
                        deg_out.at[c, pl.ds(r0, rows_per_tile)])

    return body


def _tc_combine(x, a0, a1, deg2, w_self, w_neigh, b2, blk):
    n, d = x.shape

    def body(x_ref, a0_ref, a1_ref, deg_ref, ws_ref, wn_ref, b_ref, o_ref):
        agg = a0_ref[...] + a1_ref[...]
        deg = deg_ref[...]
        degsum = jnp.maximum(deg[:, 0] + deg[:, 1], 1.0)
        h_neigh = agg / degsum[:, None]
        o_ref[...] = (
            jnp.dot(x_ref[...], ws_ref[...], preferred_element_type=jnp.float32)
            + jnp.dot(h_neigh, wn_ref[...], preferred_element_type=jnp.float32)
            + b_ref[...]
        )

    grid = (n // blk,)
    return pl.pallas_call(
        body,
        grid=grid,
        in_specs=[
            pl.BlockSpec((blk, d), lambda i: (i, 0)),
            pl.BlockSpec((blk, d), lambda i: (i, 0)),
            pl.BlockSpec((blk, d), lambda i: (i, 0)),
            pl.BlockSpec((blk, NC), lambda i: (i, 0)),
            pl.BlockSpec((d, d), lambda i: (0, 0)),
            pl.BlockSpec((d, d), lambda i: (0, 0)),
            pl.BlockSpec((1, d), lambda i: (0, 0)),
        ],
        out_specs=pl.BlockSpec((blk, d), lambda i: (i, 0)),
        out_shape=jax.ShapeDtypeStruct((n, d), jnp.float32),
    )(x, a0, a1, deg2, w_self, w_neigh, b2)


def kernel(inputs, edge_index, layer_id, n_layers, W_self, W_neigh, b):
    n, d = inputs.shape
    e = edge_index.shape[1]

    per_step = NW * CHUNK
    n_chunks = -(-e // per_step)
    e_pad = n_chunks * per_step
    n_pad = -(-(n + 1) // (NS * CHUNK)) * (NS * CHUNK)

    src = edge_index[0]
    dst = edge_index[1]
    pad = e_pad - e
    if pad:
        src = jnp.concatenate([src, jnp.zeros((pad,), jnp.int32)])
        dst = jnp.concatenate([dst, jnp.full((pad,), n, jnp.int32)])
    src_t = src.reshape(NW, n_chunks, CHUNK)
    dst_t = dst.reshape(NW, n_chunks, CHUNK)

    agg_parts, deg_parts = _sc_aggregate(n_pad, d, n_chunks)(src_t, dst_t, inputs)

    a0 = agg_parts[0, :n]
    a1 = agg_parts[1, :n]
    deg2 = deg_parts[:, :n].T
    b2 = b.reshape(1, d)
    return _tc_combine(inputs, a0, a1, deg2, W_self, W_neigh, b2, blk=1000)
